# baseline (speedup 1.0000x reference)
import functools

import jax
import jax.numpy as jnp
from jax import lax
from jax.experimental import pallas as pl
from jax.experimental.pallas import tpu as pltpu
from jax.experimental.pallas import tpu_sc as plsc

V, E = 10000, 320000
FV, FE, H = 128, 16, 128

NC, NS, L = 2, 16, 16
NW = NC * NS
EW = E // NW
B = 80
NCHUNK = EW // B
NVB = V // B
HL = H // L
UNROLL = 2

_PREC = lax.Precision.HIGHEST


def _silu(v):
    return v * jax.nn.sigmoid(v)


def _embed_body(x_ref, w_ref, b_ref, o_ref):
    o_ref[...] = _silu(
        jnp.dot(x_ref[...], w_ref[...], precision=_PREC,
                preferred_element_type=jnp.float32) + b_ref[...])


def _tc_embed(x, w, b, blk):
    n = x.shape[0]
    grid = (n // blk,)
    return pl.pallas_call(
        _embed_body,
        grid=grid,
        in_specs=[
            pl.BlockSpec((blk, x.shape[1]), lambda i: (i, 0)),
            pl.BlockSpec(w.shape, lambda i: (0, 0)),
            pl.BlockSpec((1, H), lambda i: (0, 0)),
        ],
        out_specs=pl.BlockSpec((blk, H), lambda i: (i, 0)),
        out_shape=jax.ShapeDtypeStruct((n, H), jnp.float32),
    )(x, w, b.reshape(1, H))


def _embed_t_body(xt_ref, w_ref, b_ref, o_ref):
    y = lax.dot_general(xt_ref[...], w_ref[...], (((0,), (0,)), ((), ())),
                        precision=_PREC, preferred_element_type=jnp.float32)
    o_ref[...] = _silu(y + b_ref[...])


def _tc_embed_t(xt, w, b, blk):
    n = xt.shape[1]
    grid = (n // blk,)
    return pl.pallas_call(
        _embed_t_body,
        grid=grid,
        in_specs=[
            pl.BlockSpec((xt.shape[0], blk), lambda i: (0, i)),
            pl.BlockSpec(w.shape, lambda i: (0, 0)),
            pl.BlockSpec((1, H), lambda i: (0, 0)),
        ],
        out_specs=pl.BlockSpec((blk, H), lambda i: (i, 0)),
        out_shape=jax.ShapeDtypeStruct((n, H), jnp.float32),
    )(xt, w, b.reshape(1, H))


def _update_body(parts_ref, x_ref, w_ref, b_ref, o_ref):
    agg = parts_ref[0] + parts_ref[1]
    o_ref[...] = x_ref[...] + _silu(
        jnp.dot(agg, w_ref[...], precision=_PREC,
                preferred_element_type=jnp.float32) + b_ref[...])


def _tc_update(parts, x, w, b, blk=1000):
    grid = (V // blk,)
    return pl.pallas_call(
        _update_body,
        grid=grid,
        in_specs=[
            pl.BlockSpec((NC, blk, H), lambda i: (0, i, 0)),
            pl.BlockSpec((blk, H), lambda i: (i, 0)),
            pl.BlockSpec((H, H), lambda i: (0, 0)),
            pl.BlockSpec((1, H), lambda i: (0, 0)),
        ],
        out_specs=pl.BlockSpec((blk, H), lambda i: (i, 0)),
        out_shape=jax.ShapeDtypeStruct((V, H), jnp.float32),
    )(parts, x, w, b.reshape(1, H))


def _final_body(x_ref, xi_ref, wa_ref, wb_ref, b_ref, o_ref):
    acc = jnp.dot(x_ref[...], wa_ref[...], precision=_PREC,
                  preferred_element_type=jnp.float32)
    acc += jnp.dot(xi_ref[...], wb_ref[...], precision=_PREC,
                   preferred_element_type=jnp.float32)
    o_ref[...] = _silu(acc + b_ref[...])


def _tc_final(x, x_inp, w_f, b_f, blk=1000):
    grid = (V // blk,)
    wa = w_f[:H]
    wb = w_f[H:]
    return pl.pallas_call(
        _final_body,
        grid=grid,
        in_specs=[
            pl.BlockSpec((blk, H), lambda i: (i, 0)),
            pl.BlockSpec((blk, FV), lambda i: (i, 0)),
            pl.BlockSpec((H, H), lambda i: (0, 0)),
            pl.BlockSpec((FV, H), lambda i: (0, 0)),
            pl.BlockSpec((1, H), lambda i: (0, 0)),
        ],
        out_specs=pl.BlockSpec((blk, H), lambda i: (i, 0)),
        out_shape=jax.ShapeDtypeStruct((V, H), jnp.float32),
    )(x, x_inp, wa, wb, b_f.reshape(1, H))


_mesh = plsc.VectorSubcoreMesh(core_axis_name="c", subcore_axis_name="s",
                               num_cores=NC, num_subcores=NS)

NBUF = 2


@functools.partial(
    pl.kernel,
    out_type=jax.ShapeDtypeStruct((NC, V, H), jnp.float32),
    mesh=_mesh,
    scratch_types=(
        [pltpu.VMEM((B,), jnp.int32) for _ in range(NBUF)]
        + [pltpu.VMEM((B,), jnp.int32) for _ in range(NBUF)]
        + [pltpu.VMEM((B, H), jnp.float32) for _ in range(NBUF)]
        + [pltpu.VMEM((B, H), jnp.float32) for _ in range(NBUF)]
        + [
            pltpu.VMEM_SHARED((V, H), jnp.float32),
            pltpu.SemaphoreType.DMA((NBUF,)),
            pltpu.SemaphoreType.DMA((NBUF,)),
            pltpu.SemaphoreType.DMA((NBUF,)),
            pltpu.SemaphoreType.DMA((NBUF,)),
            pltpu.SemaphoreType.DMA((NBUF,)),
        ]
    ),
)
def _sc_edge_stage(x_hbm, ea_hbm, src_hbm, dst_hbm, out_hbm, *refs):
    src_v = refs[0:NBUF]
    dst_v = refs[NBUF:2 * NBUF]
    rows_v = refs[2 * NBUF:3 * NBUF]
    eab_v = refs[3 * NBUF:4 * NBUF]
    agg_sh, sem_src, sem_dst, sem_e, sem_g, sem_s = refs[4 * NBUF:]

    c = lax.axis_index("c")
    s = lax.axis_index("s")
    w = c * NS + s
    ebase = w * EW

    zeros = jnp.zeros((L,), jnp.float32)

    @pl.loop(0, B)
    def _z(i):
        for j in range(HL):
            rows_v[0][i, pl.ds(j * L, L)] = zeros

    @pl.loop(s, NVB, step=NS)
    def _zs(k):
        pltpu.sync_copy(rows_v[0],
                        agg_sh.at[pl.ds(pl.multiple_of(k * B, 8), B)])

    plsc.subcore_barrier()


    def chunk_slice(k):
        return pl.ds(pl.multiple_of(ebase + k * B, 8), B)

    def issue_src_ea(k, b):
        base = chunk_slice(k)
        pltpu.async_copy(src_hbm.at[base], src_v[b], sem_src.at[b])
        pltpu.async_copy(ea_hbm.at[base], eab_v[b], sem_e.at[b])

    def issue_dst(k, b):
        pltpu.async_copy(dst_hbm.at[chunk_slice(k)], dst_v[b], sem_dst.at[b])

    def wait_src(b):
        pltpu.make_async_copy(src_hbm.at[pl.ds(0, B)], src_v[b],
                              sem_src.at[b]).wait()

    def issue_gather(b):
        pltpu.async_copy(x_hbm.at[src_v[b]], rows_v[b], sem_g.at[b])

    def wait_ea_gather(b):
        pltpu.make_async_copy(ea_hbm.at[pl.ds(0, B)], eab_v[b],
                              sem_e.at[b]).wait()
        pltpu.make_async_copy(x_hbm.at[src_v[b]], rows_v[b],
                              sem_g.at[b]).wait()

    def issue_scatter(b):
        pltpu.make_async_copy(dst_hbm.at[pl.ds(0, B)], dst_v[b],
                              sem_dst.at[b]).wait()
        pltpu.async_copy(rows_v[b], agg_sh.at[dst_v[b]],
                         sem_s.at[b], add=True)

    def wait_scatter(b):
        pltpu.make_async_copy(rows_v[b], agg_sh.at[dst_v[b]],
                              sem_s.at[b]).wait()

    def compute(b):
        @pl.loop(0, B, unroll=UNROLL)
        def _mul(i):
            for j in range(HL):
                sl = pl.ds(j * L, L)
                rows_v[b][i, sl] = rows_v[b][i, sl] * eab_v[b][i, sl]

    def step(k, p, scat_wait, next_k):
        q = (p + 1) % NBUF
        if next_k is not None:
            issue_src_ea(next_k, q)
        if scat_wait:
            wait_scatter(q)
        if next_k is not None:
            issue_dst(next_k, q)
            wait_src(q)
            issue_gather(q)
        wait_ea_gather(p)
        compute(p)
        issue_scatter(p)

    issue_src_ea(0, 0)
    issue_dst(0, 0)
    wait_src(0)
    issue_gather(0)

    step(0, 0, False, 1)

    @pl.loop(0, (NCHUNK - 3) // NBUF)
    def _group(g):
        k0 = 1 + g * NBUF
        for t in range(NBUF):
            k = k0 + t
            step(k, (1 + t) % NBUF, True, k + 1)

    step(NCHUNK - 2, (NCHUNK - 2) % NBUF, True, NCHUNK - 1)
    step(NCHUNK - 1, (NCHUNK - 1) % NBUF, True, None)

    wait_scatter((NCHUNK - 1) % NBUF)

    plsc.subcore_barrier()

    @pl.loop(s, NVB, step=NS)
    def _rd(k):
        off = pl.ds(pl.multiple_of(k * B, 8), B)
        pltpu.sync_copy(agg_sh.at[off], out_hbm.at[c, off])


def kernel(x_inp, edge_index, edge_attr, W_ne, b_ne, W_ee, b_ee,
           W_b0, b_b0, W_b1, b_b1, W_f, b_f):
    src = edge_index[0]
    dst = edge_index[1]

    x = _tc_embed(x_inp, W_ne, b_ne, blk=1000)
    ea = _tc_embed_t(edge_attr.T, W_ee, b_ee, blk=12800)

    for (w, b) in ((W_b0, b_b0), (W_b1, b_b1)):
        parts = _sc_edge_stage(x, ea, src, dst)
        x = _tc_update(parts, x, w, b)

    return _tc_final(x, x_inp, W_f, b_f)


# speedup vs baseline: 1.9116x; 1.9116x over previous
import functools

import jax
import jax.numpy as jnp
from jax import lax
from jax.experimental import pallas as pl
from jax.experimental.pallas import tpu as pltpu
from jax.experimental.pallas import tpu_sc as plsc

V, E = 10000, 320000
FV, FE, H = 128, 16, 128

NC, NS, L = 2, 16, 16
NW = NC * NS
EW = E // NW
B = 80
NCHUNK = EW // B
NVB = V // B
HL = H // L
UNROLL = None

_PREC = lax.Precision.HIGHEST


def _silu(v):
    return v * jax.nn.sigmoid(v)


def _embed_body(x_ref, w_ref, b_ref, o_ref):
    o_ref[...] = _silu(
        jnp.dot(x_ref[...], w_ref[...], precision=_PREC,
                preferred_element_type=jnp.float32) + b_ref[...])


def _tc_embed(x, w, b, blk):
    n = x.shape[0]
    grid = (n // blk,)
    return pl.pallas_call(
        _embed_body,
        grid=grid,
        in_specs=[
            pl.BlockSpec((blk, x.shape[1]), lambda i: (i, 0)),
            pl.BlockSpec(w.shape, lambda i: (0, 0)),
            pl.BlockSpec((1, H), lambda i: (0, 0)),
        ],
        out_specs=pl.BlockSpec((blk, H), lambda i: (i, 0)),
        out_shape=jax.ShapeDtypeStruct((n, H), jnp.float32),
    )(x, w, b.reshape(1, H))


def _embed_t_body(xt_ref, w_ref, b_ref, o_ref):
    y = lax.dot_general(xt_ref[...], w_ref[...], (((0,), (0,)), ((), ())),
                        precision=_PREC, preferred_element_type=jnp.float32)
    o_ref[...] = _silu(y + b_ref[...])


def _tc_embed_t(xt, w, b, blk):
    n = xt.shape[1]
    grid = (n // blk,)
    return pl.pallas_call(
        _embed_t_body,
        grid=grid,
        in_specs=[
            pl.BlockSpec((xt.shape[0], blk), lambda i: (0, i)),
            pl.BlockSpec(w.shape, lambda i: (0, 0)),
            pl.BlockSpec((1, H), lambda i: (0, 0)),
        ],
        out_specs=pl.BlockSpec((blk, H), lambda i: (i, 0)),
        out_shape=jax.ShapeDtypeStruct((n, H), jnp.float32),
    )(xt, w, b.reshape(1, H))


def _update_body(parts_ref, x_ref, w_ref, b_ref, o_ref):
    agg = parts_ref[0] + parts_ref[1]
    o_ref[...] = x_ref[...] + _silu(
        jnp.dot(agg, w_ref[...], precision=_PREC,
                preferred_element_type=jnp.float32) + b_ref[...])


def _tc_update(parts, x, w, b, blk=1000):
    grid = (V // blk,)
    return pl.pallas_call(
        _update_body,
        grid=grid,
        in_specs=[
            pl.BlockSpec((NC, blk, H), lambda i: (0, i, 0)),
            pl.BlockSpec((blk, H), lambda i: (i, 0)),
            pl.BlockSpec((H, H), lambda i: (0, 0)),
            pl.BlockSpec((1, H), lambda i: (0, 0)),
        ],
        out_specs=pl.BlockSpec((blk, H), lambda i: (i, 0)),
        out_shape=jax.ShapeDtypeStruct((V, H), jnp.float32),
    )(parts, x, w, b.reshape(1, H))


def _final_body(x_ref, xi_ref, wa_ref, wb_ref, b_ref, o_ref):
    acc = jnp.dot(x_ref[...], wa_ref[...], precision=_PREC,
                  preferred_element_type=jnp.float32)
    acc += jnp.dot(xi_ref[...], wb_ref[...], precision=_PREC,
                   preferred_element_type=jnp.float32)
    o_ref[...] = _silu(acc + b_ref[...])


def _tc_final(x, x_inp, w_f, b_f, blk=1000):
    grid = (V // blk,)
    wa = w_f[:H]
    wb = w_f[H:]
    return pl.pallas_call(
        _final_body,
        grid=grid,
        in_specs=[
            pl.BlockSpec((blk, H), lambda i: (i, 0)),
            pl.BlockSpec((blk, FV), lambda i: (i, 0)),
            pl.BlockSpec((H, H), lambda i: (0, 0)),
            pl.BlockSpec((FV, H), lambda i: (0, 0)),
            pl.BlockSpec((1, H), lambda i: (0, 0)),
        ],
        out_specs=pl.BlockSpec((blk, H), lambda i: (i, 0)),
        out_shape=jax.ShapeDtypeStruct((V, H), jnp.float32),
    )(x, x_inp, wa, wb, b_f.reshape(1, H))


_mesh = plsc.VectorSubcoreMesh(core_axis_name="c", subcore_axis_name="s",
                               num_cores=NC, num_subcores=NS)

NBUF = 2


@functools.partial(
    pl.kernel,
    out_type=jax.ShapeDtypeStruct((NC, V, H), jnp.float32),
    mesh=_mesh,
    scratch_types=(
        [pltpu.VMEM((B,), jnp.int32) for _ in range(NBUF)]
        + [pltpu.VMEM((B,), jnp.int32) for _ in range(NBUF)]
        + [pltpu.VMEM((B, H), jnp.float32) for _ in range(NBUF)]
        + [pltpu.VMEM((B, H), jnp.float32) for _ in range(NBUF)]
        + [
            pltpu.VMEM_SHARED((V, H), jnp.float32),
            pltpu.SemaphoreType.DMA((NBUF,)),
            pltpu.SemaphoreType.DMA((NBUF,)),
            pltpu.SemaphoreType.DMA((NBUF,)),
            pltpu.SemaphoreType.DMA((NBUF,)),
            pltpu.SemaphoreType.DMA((NBUF,)),
        ]
    ),
)
def _sc_edge_stage(x_hbm, ea_hbm, src_hbm, dst_hbm, out_hbm, *refs):
    src_v = refs[0:NBUF]
    dst_v = refs[NBUF:2 * NBUF]
    rows_v = refs[2 * NBUF:3 * NBUF]
    eab_v = refs[3 * NBUF:4 * NBUF]
    agg_sh, sem_src, sem_dst, sem_e, sem_g, sem_s = refs[4 * NBUF:]

    c = lax.axis_index("c")
    s = lax.axis_index("s")
    w = c * NS + s
    ebase = w * EW

    zeros = jnp.zeros((L,), jnp.float32)

    @pl.loop(0, B)
    def _z(i):
        for j in range(HL):
            rows_v[0][i, pl.ds(j * L, L)] = zeros

    @pl.loop(s, NVB, step=NS)
    def _zs(k):
        pltpu.sync_copy(rows_v[0],
                        agg_sh.at[pl.ds(pl.multiple_of(k * B, 8), B)])

    plsc.subcore_barrier()


    def chunk_slice(k):
        return pl.ds(pl.multiple_of(ebase + k * B, 8), B)

    def issue_src_ea(k, b):
        base = chunk_slice(k)
        pltpu.async_copy(src_hbm.at[base], src_v[b], sem_src.at[b])
        pltpu.async_copy(ea_hbm.at[base], eab_v[b], sem_e.at[b])

    def issue_dst(k, b):
        pltpu.async_copy(dst_hbm.at[chunk_slice(k)], dst_v[b], sem_dst.at[b])

    def wait_src(b):
        pltpu.make_async_copy(src_hbm.at[pl.ds(0, B)], src_v[b],
                              sem_src.at[b]).wait()

    def issue_gather(b):
        pltpu.async_copy(x_hbm.at[src_v[b]], rows_v[b], sem_g.at[b])

    def wait_ea_gather(b):
        pltpu.make_async_copy(ea_hbm.at[pl.ds(0, B)], eab_v[b],
                              sem_e.at[b]).wait()
        pltpu.make_async_copy(x_hbm.at[src_v[b]], rows_v[b],
                              sem_g.at[b]).wait()

    def issue_scatter(b):
        pltpu.make_async_copy(dst_hbm.at[pl.ds(0, B)], dst_v[b],
                              sem_dst.at[b]).wait()
        pltpu.async_copy(rows_v[b], agg_sh.at[dst_v[b]],
                         sem_s.at[b], add=True)

    def wait_scatter(b):
        pltpu.make_async_copy(rows_v[b], agg_sh.at[dst_v[b]],
                              sem_s.at[b]).wait()

    def compute(b):
        @pl.loop(0, B, unroll=UNROLL)
        def _mul(i):
            for j in range(HL):
                sl = pl.ds(j * L, L)
                rows_v[b][i, sl] = rows_v[b][i, sl] * eab_v[b][i, sl]

    def step(k, p, scat_wait, next_k):
        q = (p + 1) % NBUF
        if next_k is not None:
            issue_src_ea(next_k, q)
        if scat_wait:
            wait_scatter(q)
        if next_k is not None:
            issue_dst(next_k, q)
            wait_src(q)
            issue_gather(q)
        wait_ea_gather(p)
        compute(p)
        issue_scatter(p)

    issue_src_ea(0, 0)
    issue_dst(0, 0)
    wait_src(0)
    issue_gather(0)

    step(0, 0, False, 1)

    @pl.loop(0, (NCHUNK - 3) // NBUF)
    def _group(g):
        k0 = 1 + g * NBUF
        for t in range(NBUF):
            k = k0 + t
            step(k, (1 + t) % NBUF, True, k + 1)

    step(NCHUNK - 2, (NCHUNK - 2) % NBUF, True, NCHUNK - 1)
    step(NCHUNK - 1, (NCHUNK - 1) % NBUF, True, None)

    wait_scatter((NCHUNK - 1) % NBUF)

    plsc.subcore_barrier()

    @pl.loop(s, NVB, step=NS)
    def _rd(k):
        off = pl.ds(pl.multiple_of(k * B, 8), B)
        pltpu.sync_copy(agg_sh.at[off], out_hbm.at[c, off])


def kernel(x_inp, edge_index, edge_attr, W_ne, b_ne, W_ee, b_ee,
           W_b0, b_b0, W_b1, b_b1, W_f, b_f):
    src = edge_index[0]
    dst = edge_index[1]

    x = _tc_embed(x_inp, W_ne, b_ne, blk=1000)
    ea = _tc_embed_t(edge_attr.T, W_ee, b_ee, blk=12800)

    for (w, b) in ((W_b0, b_b0), (W_b1, b_b1)):
        parts = _sc_edge_stage(x, ea, src, dst)
        x = _tc_update(parts, x, w, b)

    return _tc_final(x, x_inp, W_f, b_f)


# speedup vs baseline: 1.9326x; 1.0110x over previous
import functools

import jax
import jax.numpy as jnp
from jax import lax
from jax.experimental import pallas as pl
from jax.experimental.pallas import tpu as pltpu
from jax.experimental.pallas import tpu_sc as plsc

V, E = 10000, 320000
FV, FE, H = 128, 16, 128

NC, NS, L = 2, 16, 16
NW = NC * NS
EW = E // NW
B = 80
NCHUNK = EW // B
NVB = V // B
HL = H // L
UNROLL = None

_PREC = lax.Precision.HIGHEST


def _silu(v):
    return v * jax.nn.sigmoid(v)


def _embed_body(x_ref, w_ref, b_ref, o_ref):
    o_ref[...] = _silu(
        jnp.dot(x_ref[...], w_ref[...], precision=_PREC,
                preferred_element_type=jnp.float32) + b_ref[...])


def _tc_embed(x, w, b, blk):
    n = x.shape[0]
    grid = (n // blk,)
    return pl.pallas_call(
        _embed_body,
        grid=grid,
        in_specs=[
            pl.BlockSpec((blk, x.shape[1]), lambda i: (i, 0)),
            pl.BlockSpec(w.shape, lambda i: (0, 0)),
            pl.BlockSpec((1, H), lambda i: (0, 0)),
        ],
        out_specs=pl.BlockSpec((blk, H), lambda i: (i, 0)),
        out_shape=jax.ShapeDtypeStruct((n, H), jnp.float32),
    )(x, w, b.reshape(1, H))


def _embed_t_body(xt_ref, w_ref, b_ref, o_ref):
    y = lax.dot_general(xt_ref[...], w_ref[...], (((0,), (0,)), ((), ())),
                        precision=_PREC, preferred_element_type=jnp.float32)
    o_ref[...] = _silu(y + b_ref[...])


def _tc_embed_t(xt, w, b, blk):
    n = xt.shape[1]
    grid = (n // blk,)
    return pl.pallas_call(
        _embed_t_body,
        grid=grid,
        in_specs=[
            pl.BlockSpec((xt.shape[0], blk), lambda i: (0, i)),
            pl.BlockSpec(w.shape, lambda i: (0, 0)),
            pl.BlockSpec((1, H), lambda i: (0, 0)),
        ],
        out_specs=pl.BlockSpec((blk, H), lambda i: (i, 0)),
        out_shape=jax.ShapeDtypeStruct((n, H), jnp.float32),
    )(xt, w, b.reshape(1, H))


def _update_body(parts_ref, x_ref, w_ref, b_ref, o_ref):
    agg = parts_ref[0] + parts_ref[1]
    o_ref[...] = x_ref[...] + _silu(
        jnp.dot(agg, w_ref[...], precision=_PREC,
                preferred_element_type=jnp.float32) + b_ref[...])


def _tc_update(parts, x, w, b, blk=1000):
    grid = (V // blk,)
    return pl.pallas_call(
        _update_body,
        grid=grid,
        in_specs=[
            pl.BlockSpec((NC, blk, H), lambda i: (0, i, 0)),
            pl.BlockSpec((blk, H), lambda i: (i, 0)),
            pl.BlockSpec((H, H), lambda i: (0, 0)),
            pl.BlockSpec((1, H), lambda i: (0, 0)),
        ],
        out_specs=pl.BlockSpec((blk, H), lambda i: (i, 0)),
        out_shape=jax.ShapeDtypeStruct((V, H), jnp.float32),
    )(parts, x, w, b.reshape(1, H))


def _final_body(x_ref, xi_ref, wa_ref, wb_ref, b_ref, o_ref):
    acc = jnp.dot(x_ref[...], wa_ref[...], precision=_PREC,
                  preferred_element_type=jnp.float32)
    acc += jnp.dot(xi_ref[...], wb_ref[...], precision=_PREC,
                   preferred_element_type=jnp.float32)
    o_ref[...] = _silu(acc + b_ref[...])


def _tc_final(x, x_inp, w_f, b_f, blk=1000):
    grid = (V // blk,)
    wa = w_f[:H]
    wb = w_f[H:]
    return pl.pallas_call(
        _final_body,
        grid=grid,
        in_specs=[
            pl.BlockSpec((blk, H), lambda i: (i, 0)),
            pl.BlockSpec((blk, FV), lambda i: (i, 0)),
            pl.BlockSpec((H, H), lambda i: (0, 0)),
            pl.BlockSpec((FV, H), lambda i: (0, 0)),
            pl.BlockSpec((1, H), lambda i: (0, 0)),
        ],
        out_specs=pl.BlockSpec((blk, H), lambda i: (i, 0)),
        out_shape=jax.ShapeDtypeStruct((V, H), jnp.float32),
    )(x, x_inp, wa, wb, b_f.reshape(1, H))


_mesh = plsc.VectorSubcoreMesh(core_axis_name="c", subcore_axis_name="s",
                               num_cores=NC, num_subcores=NS)

NBUF = 2


@functools.partial(
    pl.kernel,
    out_type=jax.ShapeDtypeStruct((NC, V, H), jnp.float32),
    mesh=_mesh,
    scratch_types=(
        [pltpu.VMEM((B,), jnp.int32) for _ in range(NBUF)]
        + [pltpu.VMEM((B,), jnp.int32) for _ in range(NBUF)]
        + [pltpu.VMEM((B, H), jnp.float32) for _ in range(NBUF)]
        + [pltpu.VMEM((B, H), jnp.float32) for _ in range(NBUF)]
        + [
            pltpu.VMEM_SHARED((V, H), jnp.float32),
            pltpu.SemaphoreType.DMA((NBUF,)),
            pltpu.SemaphoreType.DMA((NBUF,)),
            pltpu.SemaphoreType.DMA((NBUF,)),
            pltpu.SemaphoreType.DMA((NBUF,)),
            pltpu.SemaphoreType.DMA((NBUF,)),
        ]
    ),
)
def _sc_edge_stage(x_hbm, ea_hbm, src_hbm, dst_hbm, out_hbm, *refs):
    src_v = refs[0:NBUF]
    dst_v = refs[NBUF:2 * NBUF]
    rows_v = refs[2 * NBUF:3 * NBUF]
    eab_v = refs[3 * NBUF:4 * NBUF]
    agg_sh, sem_src, sem_dst, sem_e, sem_g, sem_s = refs[4 * NBUF:]

    c = lax.axis_index("c")
    s = lax.axis_index("s")
    w = c * NS + s
    ebase = w * EW

    zeros = jnp.zeros((L,), jnp.float32)

    @pl.loop(0, B)
    def _z(i):
        for j in range(HL):
            rows_v[0][i, pl.ds(j * L, L)] = zeros

    @pl.loop(s, NVB, step=NS)
    def _zs(k):
        pltpu.sync_copy(rows_v[0],
                        agg_sh.at[pl.ds(pl.multiple_of(k * B, 8), B)])

    plsc.subcore_barrier()


    def chunk_slice(k):
        return pl.ds(pl.multiple_of(ebase + k * B, 8), B)

    def issue_src_ea(k, b):
        base = chunk_slice(k)
        pltpu.async_copy(src_hbm.at[base], src_v[b], sem_src.at[b])
        pltpu.async_copy(ea_hbm.at[base], eab_v[b], sem_e.at[b])

    def issue_dst(k, b):
        pltpu.async_copy(dst_hbm.at[chunk_slice(k)], dst_v[b], sem_dst.at[b])

    def wait_src(b):
        pltpu.make_async_copy(src_hbm.at[pl.ds(0, B)], src_v[b],
                              sem_src.at[b]).wait()

    def issue_gather(b):
        pltpu.async_copy(x_hbm.at[src_v[b]], rows_v[b], sem_g.at[b])

    def wait_ea_gather(b):
        pltpu.make_async_copy(ea_hbm.at[pl.ds(0, B)], eab_v[b],
                              sem_e.at[b]).wait()
        pltpu.make_async_copy(x_hbm.at[src_v[b]], rows_v[b],
                              sem_g.at[b]).wait()

    def issue_scatter(b):
        pltpu.make_async_copy(dst_hbm.at[pl.ds(0, B)], dst_v[b],
                              sem_dst.at[b]).wait()
        pltpu.async_copy(rows_v[b], agg_sh.at[dst_v[b]],
                         sem_s.at[b], add=True)

    def wait_scatter(b):
        pltpu.make_async_copy(rows_v[b], agg_sh.at[dst_v[b]],
                              sem_s.at[b]).wait()

    def compute(b):
        @pl.loop(0, B, unroll=UNROLL)
        def _mul(i):
            for j in range(HL):
                sl = pl.ds(j * L, L)
                rows_v[b][i, sl] = rows_v[b][i, sl] * eab_v[b][i, sl]

    def step(k, p, scat_wait, next_k):
        q = (p + 1) % NBUF
        if next_k is not None:
            issue_src_ea(next_k, q)
        if scat_wait:
            wait_scatter(q)
        if next_k is not None:
            issue_dst(next_k, q)
            wait_src(q)
            issue_gather(q)
        wait_ea_gather(p)
        compute(p)
        issue_scatter(p)

    issue_src_ea(0, 0)
    issue_dst(0, 0)
    wait_src(0)
    issue_gather(0)

    step(0, 0, False, 1)

    @pl.loop(0, (NCHUNK - 3) // NBUF)
    def _group(g):
        k0 = 1 + g * NBUF
        for t in range(NBUF):
            k = k0 + t
            step(k, (1 + t) % NBUF, True, k + 1)

    step(NCHUNK - 2, (NCHUNK - 2) % NBUF, True, NCHUNK - 1)
    step(NCHUNK - 1, (NCHUNK - 1) % NBUF, True, None)

    wait_scatter((NCHUNK - 1) % NBUF)

    plsc.subcore_barrier()

    @pl.loop(s, NVB, step=NS)
    def _rd(k):
        off = pl.ds(pl.multiple_of(k * B, 8), B)
        pltpu.sync_copy(agg_sh.at[off], out_hbm.at[c, off])


def _update_final_body(parts_ref, x_ref, xi_ref, w_ref, b_ref,
                       wa_ref, wb_ref, bf_ref, o_ref):
    agg = parts_ref[0] + parts_ref[1]
    x2 = x_ref[...] + _silu(
        jnp.dot(agg, w_ref[...], precision=_PREC,
                preferred_element_type=jnp.float32) + b_ref[...])
    acc = jnp.dot(x2, wa_ref[...], precision=_PREC,
                  preferred_element_type=jnp.float32)
    acc += jnp.dot(xi_ref[...], wb_ref[...], precision=_PREC,
                   preferred_element_type=jnp.float32)
    o_ref[...] = _silu(acc + bf_ref[...])


def _tc_update_final(parts, x, x_inp, w, b, w_f, b_f, blk=1000):
    grid = (V // blk,)
    wa = w_f[:H]
    wb = w_f[H:]
    return pl.pallas_call(
        _update_final_body,
        grid=grid,
        in_specs=[
            pl.BlockSpec((NC, blk, H), lambda i: (0, i, 0)),
            pl.BlockSpec((blk, H), lambda i: (i, 0)),
            pl.BlockSpec((blk, FV), lambda i: (i, 0)),
            pl.BlockSpec((H, H), lambda i: (0, 0)),
            pl.BlockSpec((1, H), lambda i: (0, 0)),
            pl.BlockSpec((H, H), lambda i: (0, 0)),
            pl.BlockSpec((FV, H), lambda i: (0, 0)),
            pl.BlockSpec((1, H), lambda i: (0, 0)),
        ],
        out_specs=pl.BlockSpec((blk, H), lambda i: (i, 0)),
        out_shape=jax.ShapeDtypeStruct((V, H), jnp.float32),
    )(parts, x, x_inp, w, b.reshape(1, H), wa, wb, b_f.reshape(1, H))


def kernel(x_inp, edge_index, edge_attr, W_ne, b_ne, W_ee, b_ee,
           W_b0, b_b0, W_b1, b_b1, W_f, b_f):
    src = edge_index[0]
    dst = edge_index[1]

    x = _tc_embed(x_inp, W_ne, b_ne, blk=1000)
    ea = _tc_embed_t(edge_attr.T, W_ee, b_ee, blk=12800)

    parts = _sc_edge_stage(x, ea, src, dst)
    x = _tc_update(parts, x, W_b0, b_b0)
    parts = _sc_edge_stage(x, ea, src, dst)
    return _tc_update_final(parts, x, x_inp, W_b1, b_b1, W_f, b_f)
